# baseline (speedup 1.0000x reference)
import functools

import jax
import jax.numpy as jnp
from jax import lax
from jax.experimental import pallas as pl
from jax.experimental.pallas import tpu as pltpu
from jax.experimental.pallas import tpu_sc as plsc

_LANES = 16
_IDX_CHUNK = 128


def _sc_pool(x_flat2d, emb_table, B, CTX, E):
    NC, NS = 2, 16
    NW = NC * NS
    b_per_w = B // NW
    idx_per_w = b_per_w * CTX
    n_chunks = idx_per_w // _IDX_CHUNK
    inv = jnp.float32(1.0 / CTX)

    mesh = plsc.VectorSubcoreMesh(core_axis_name="c", subcore_axis_name="s")

    @functools.partial(
        pl.kernel,
        out_type=jax.ShapeDtypeStruct((B, E), jnp.float32),
        mesh=mesh,
        scratch_types=[
            pltpu.VMEM((n_chunks, _IDX_CHUNK), jnp.int32),
            pltpu.VMEM((idx_per_w, E), jnp.float32),
            pltpu.VMEM((b_per_w, E), jnp.float32),
            pltpu.SemaphoreType.DMA,
        ],
        compiler_params=pltpu.CompilerParams(use_tc_tiling_on_sc=False),
    )
    def pool(x_hbm, tbl_hbm, out_hbm, idx_v, rows_v, pooled_v, sem):
        wid = lax.axis_index("s") * NC + lax.axis_index("c")
        pltpu.sync_copy(x_hbm.at[wid], idx_v)
        copies = []
        for j in range(n_chunks):
            copies.append(pltpu.async_copy(
                tbl_hbm.at[idx_v.at[j]],
                rows_v.at[pl.ds(j * _IDX_CHUNK, _IDX_CHUNK)],
                sem,
            ))
        for c in copies:
            c.wait()

        def body(i, carry):
            base = i * CTX
            for cch in range(E // _LANES):
                sl = pl.ds(cch * _LANES, _LANES)
                acc = rows_v[base, sl]
                for j in range(1, CTX):
                    acc = acc + rows_v[base + j, sl]
                pooled_v[i, sl] = acc * inv
            return carry

        lax.fori_loop(0, b_per_w, body, 0)
        pltpu.sync_copy(pooled_v, out_hbm.at[pl.ds(wid * b_per_w, b_per_w)])

    return pool(x_flat2d, emb_table)


def _tc_project(pooled, W, b2d, NV=2048):
    B, E = pooled.shape
    V = W.shape[0]
    grid = pl.cdiv(V, NV)

    def mm(p_ref, w_ref, b_ref, o_ref):
        o_ref[...] = lax.dot_general(
            p_ref[...], w_ref[...],
            (((1,), (1,)), ((), ())),
            preferred_element_type=jnp.float32,
        ) + b_ref[...]

    return pl.pallas_call(
        mm,
        grid=(grid,),
        in_specs=[
            pl.BlockSpec((B, E), lambda i: (0, 0)),
            pl.BlockSpec((NV, E), lambda i: (i, 0)),
            pl.BlockSpec((1, NV), lambda i: (0, i)),
        ],
        out_specs=pl.BlockSpec((B, NV), lambda i: (0, i)),
        out_shape=jax.ShapeDtypeStruct((B, V), jnp.float32),
        compiler_params=pltpu.CompilerParams(
            dimension_semantics=("arbitrary",),
        ),
    )(pooled, W, b2d)


def kernel(x, emb_table, W, b):
    B, CTX = x.shape
    V, E = emb_table.shape
    NW = 32
    x_flat2d = x.astype(jnp.int32).reshape(
        NW, B * CTX // (NW * _IDX_CHUNK), _IDX_CHUNK)
    pooled = _sc_pool(x_flat2d, emb_table, B, CTX, E)
    return _tc_project(pooled, W, b.reshape(1, V))


# speedup vs baseline: 2.7512x; 2.7512x over previous
import functools

import jax
import jax.numpy as jnp
from jax import lax
from jax.experimental import pallas as pl
from jax.experimental.pallas import tpu as pltpu
from jax.experimental.pallas import tpu_sc as plsc

_LANES = 16
_IDX_CHUNK = 128


def _sc_pool(x_flat2d, emb_table, B, CTX, E):
    NC, NS = 2, 16
    NW = NC * NS
    b_per_w = B // NW
    idx_per_w = b_per_w * CTX
    n_chunks = idx_per_w // _IDX_CHUNK
    inv = jnp.float32(1.0 / CTX)

    mesh = plsc.VectorSubcoreMesh(core_axis_name="c", subcore_axis_name="s")

    @functools.partial(
        pl.kernel,
        out_type=jax.ShapeDtypeStruct((B, E), jnp.float32),
        mesh=mesh,
        scratch_types=[
            pltpu.VMEM((n_chunks, _IDX_CHUNK), jnp.int32),
            pltpu.VMEM((idx_per_w, E), jnp.float32),
            pltpu.VMEM((b_per_w, E), jnp.float32),
            pltpu.SemaphoreType.DMA,
        ],
        compiler_params=pltpu.CompilerParams(use_tc_tiling_on_sc=False),
    )
    def pool(x_hbm, tbl_hbm, out_hbm, idx_v, rows_v, pooled_v, sem):
        wid = lax.axis_index("s") * NC + lax.axis_index("c")
        pltpu.sync_copy(x_hbm.at[wid], idx_v)
        copies = []
        for j in range(n_chunks):
            copies.append(pltpu.async_copy(
                tbl_hbm.at[idx_v.at[j]],
                rows_v.at[pl.ds(j * _IDX_CHUNK, _IDX_CHUNK)],
                sem,
            ))
        for c in copies:
            c.wait()

        def body(i, carry):
            base = i * CTX
            for cch in range(E // _LANES):
                sl = pl.ds(cch * _LANES, _LANES)
                acc = rows_v[base, sl]
                for j in range(1, CTX):
                    acc = acc + rows_v[base + j, sl]
                pooled_v[i, sl] = acc * inv
            return carry

        lax.fori_loop(0, b_per_w, body, 0)
        pltpu.sync_copy(pooled_v, out_hbm.at[pl.ds(wid * b_per_w, b_per_w)])

    return pool(x_flat2d, emb_table)


def _tc_project_t(pooled, w_t, b_row, NV=2048):
    B, E = pooled.shape
    V = w_t.shape[1]
    grid = pl.cdiv(V, NV)

    def mm(p_ref, w_ref, b_ref, o_ref):
        dot = lax.dot_general(
            w_ref[...], p_ref[...],
            (((0,), (1,)), ((), ())),
            preferred_element_type=jnp.float32,
        )
        o_ref[...] = dot + jnp.transpose(b_ref[...])

    return pl.pallas_call(
        mm,
        grid=(grid,),
        in_specs=[
            pl.BlockSpec((B, E), lambda i: (0, 0)),
            pl.BlockSpec((E, NV), lambda i: (0, i)),
            pl.BlockSpec((1, NV), lambda i: (0, i)),
        ],
        out_specs=pl.BlockSpec((NV, B), lambda i: (i, 0)),
        out_shape=jax.ShapeDtypeStruct((V, B), jnp.float32),
        compiler_params=pltpu.CompilerParams(
            dimension_semantics=("arbitrary",),
        ),
    )(pooled, w_t, b_row)


def kernel(x, emb_table, W, b):
    B, CTX = x.shape
    V, E = emb_table.shape
    NW = 32
    x_flat2d = x.astype(jnp.int32).reshape(
        NW, B * CTX // (NW * _IDX_CHUNK), _IDX_CHUNK)
    pooled = _sc_pool(x_flat2d, emb_table, B, CTX, E)
    out_t = _tc_project_t(pooled, W.T, b.reshape(1, V))
    return out_t.T


# speedup vs baseline: 2.7712x; 1.0073x over previous
import functools

import jax
import jax.numpy as jnp
from jax import lax
from jax.experimental import pallas as pl
from jax.experimental.pallas import tpu as pltpu
from jax.experimental.pallas import tpu_sc as plsc

_LANES = 16
_IDX_CHUNK = 128


def _sc_pool(x_t, emb_table, B, CTX, E):
    NC, NS = 2, 16
    NW = NC * NS
    b_per_w = B // NW
    inv = jnp.float32(1.0 / CTX)

    mesh = plsc.VectorSubcoreMesh(core_axis_name="c", subcore_axis_name="s")

    @functools.partial(
        pl.kernel,
        out_type=jax.ShapeDtypeStruct((B, E), jnp.float32),
        mesh=mesh,
        scratch_types=[
            pltpu.VMEM((CTX, b_per_w), jnp.int32),
            pltpu.VMEM((b_per_w, E), jnp.float32),
            pltpu.SemaphoreType.DMA,
        ],
        compiler_params=pltpu.CompilerParams(use_tc_tiling_on_sc=False),
    )
    def pool(x_hbm, tbl_hbm, out_hbm, idx_v, acc_v, sem):
        wid = lax.axis_index("s") * NC + lax.axis_index("c")
        base = wid * b_per_w
        pltpu.sync_copy(x_hbm.at[:, pl.ds(base, b_per_w)], idx_v)
        zeros = jnp.zeros((_LANES,), jnp.float32)
        def zbody(i, carry):
            for cch in range(E // _LANES):
                acc_v[i, pl.ds(cch * _LANES, _LANES)] = zeros
            return carry
        lax.fori_loop(0, b_per_w, zbody, 0)
        copies = []
        for j in range(CTX):
            copies.append(pltpu.async_copy(
                tbl_hbm.at[idx_v.at[j]], acc_v, sem, add=True,
            ))
        for c in copies:
            c.wait()
        def sbody(i, carry):
            for cch in range(E // _LANES):
                sl = pl.ds(cch * _LANES, _LANES)
                acc_v[i, sl] = acc_v[i, sl] * inv
            return carry
        lax.fori_loop(0, b_per_w, sbody, 0)
        pltpu.sync_copy(acc_v, out_hbm.at[pl.ds(base, b_per_w)])

    return pool(x_t, emb_table)


def _tc_project_t(pooled, w_t, b_row, NV=2048):
    B, E = pooled.shape
    V = w_t.shape[1]
    grid = pl.cdiv(V, NV)

    def mm(p_ref, w_ref, b_ref, o_ref):
        dot = lax.dot_general(
            w_ref[...], p_ref[...],
            (((0,), (1,)), ((), ())),
            preferred_element_type=jnp.float32,
        )
        o_ref[...] = dot + jnp.transpose(b_ref[...])

    return pl.pallas_call(
        mm,
        grid=(grid,),
        in_specs=[
            pl.BlockSpec((B, E), lambda i: (0, 0)),
            pl.BlockSpec((E, NV), lambda i: (0, i)),
            pl.BlockSpec((1, NV), lambda i: (0, i)),
        ],
        out_specs=pl.BlockSpec((NV, B), lambda i: (i, 0)),
        out_shape=jax.ShapeDtypeStruct((V, B), jnp.float32),
        compiler_params=pltpu.CompilerParams(
            dimension_semantics=("arbitrary",),
        ),
    )(pooled, w_t, b_row)


def kernel(x, emb_table, W, b):
    B, CTX = x.shape
    V, E = emb_table.shape
    pooled = _sc_pool(x.astype(jnp.int32).T, emb_table, B, CTX, E)
    out_t = _tc_project_t(pooled, W.T, b.reshape(1, V))
    return out_t.T


# speedup vs baseline: 2.9983x; 1.0820x over previous
import functools

import jax
import jax.numpy as jnp
from jax import lax
from jax.experimental import pallas as pl
from jax.experimental.pallas import tpu as pltpu
from jax.experimental.pallas import tpu_sc as plsc

_LANES = 16
_IDX_CHUNK = 128
_NVC = 2048
_NVC_LOG2 = 11


def _sc_pool(x_t, emb_table, B, CTX, E):
    NC, NS = 2, 16
    NW = NC * NS
    b_per_w = B // NW
    inv = jnp.float32(1.0 / CTX)

    mesh = plsc.VectorSubcoreMesh(core_axis_name="c", subcore_axis_name="s")

    @functools.partial(
        pl.kernel,
        out_type=jax.ShapeDtypeStruct((B, E), jnp.float32),
        mesh=mesh,
        scratch_types=[
            pltpu.VMEM((CTX, b_per_w), jnp.int32),
            pltpu.VMEM((b_per_w, E), jnp.float32),
            pltpu.SemaphoreType.DMA,
        ],
        compiler_params=pltpu.CompilerParams(use_tc_tiling_on_sc=False),
    )
    def pool(x_hbm, tbl_hbm, out_hbm, idx_v, acc_v, sem):
        wid = lax.axis_index("s") * NC + lax.axis_index("c")
        base = wid * b_per_w
        pltpu.sync_copy(x_hbm.at[:, pl.ds(base, b_per_w)], idx_v)
        for j in range(CTX):
            for cch in range(b_per_w // _LANES):
                sl = pl.ds(cch * _LANES, _LANES)
                v = idx_v[j, sl]
                hi = v & jnp.int32(-_NVC)
                lo = lax.shift_left(v & jnp.int32(_NVC // 2 - 1),
                                    jnp.int32(1))
                bit = lax.shift_right_logical(v, jnp.int32(_NVC_LOG2 - 1))
                idx_v[j, sl] = hi | lo | (bit & jnp.int32(1))
        zeros = jnp.zeros((_LANES,), jnp.float32)
        def zbody(i, carry):
            for cch in range(E // _LANES):
                acc_v[i, pl.ds(cch * _LANES, _LANES)] = zeros
            return carry
        lax.fori_loop(0, b_per_w, zbody, 0)
        copies = []
        for j in range(CTX):
            copies.append(pltpu.async_copy(
                tbl_hbm.at[idx_v.at[j]], acc_v, sem, add=True,
            ))
        for c in copies:
            c.wait()
        def sbody(i, carry):
            for cch in range(E // _LANES):
                sl = pl.ds(cch * _LANES, _LANES)
                acc_v[i, sl] = acc_v[i, sl] * inv
            return carry
        lax.fori_loop(0, b_per_w, sbody, 0)
        pltpu.sync_copy(acc_v, out_hbm.at[pl.ds(base, b_per_w)])

    return pool(x_t, emb_table)


def _tc_detile(emb_t, NVc=_NVC):
    E, V = emb_t.shape
    grid = pl.cdiv(V, NVc)

    def det(in_ref, o_ref):
        t = jnp.transpose(in_ref[...])
        o_ref[...] = jnp.concatenate(
            [t[0:NVc // 2], t[NVc // 2:NVc]], axis=1)

    return pl.pallas_call(
        det,
        grid=(grid,),
        in_specs=[pl.BlockSpec((E, NVc), lambda i: (0, i))],
        out_specs=pl.BlockSpec((NVc * E // 128, 128), lambda i: (i, 0)),
        out_shape=jax.ShapeDtypeStruct((grid * NVc * E // 128, 128),
                                       jnp.float32),
        compiler_params=pltpu.CompilerParams(
            dimension_semantics=("arbitrary",),
        ),
    )(emb_t)


def _tc_project_t(pooled, w_t, b_row, NV=2048):
    B, E = pooled.shape
    V = w_t.shape[1]
    grid = pl.cdiv(V, NV)

    def mm(p_ref, w_ref, b_ref, o_ref):
        dot = lax.dot_general(
            w_ref[...], p_ref[...],
            (((0,), (1,)), ((), ())),
            preferred_element_type=jnp.float32,
        )
        o_ref[...] = dot + jnp.transpose(b_ref[...])

    return pl.pallas_call(
        mm,
        grid=(grid,),
        in_specs=[
            pl.BlockSpec((B, E), lambda i: (0, 0)),
            pl.BlockSpec((E, NV), lambda i: (0, i)),
            pl.BlockSpec((1, NV), lambda i: (0, i)),
        ],
        out_specs=pl.BlockSpec((NV, B), lambda i: (i, 0)),
        out_shape=jax.ShapeDtypeStruct((V, B), jnp.float32),
        compiler_params=pltpu.CompilerParams(
            dimension_semantics=("arbitrary",),
        ),
    )(pooled, w_t, b_row)


def kernel(x, emb_table, W, b):
    B, CTX = x.shape
    V, E = emb_table.shape
    emb_lin = _tc_detile(emb_table.T).reshape(-1, E)
    pooled = _sc_pool(x.astype(jnp.int32).T, emb_lin, B, CTX, E)
    out_t = _tc_project_t(pooled, W.T, b.reshape(1, V))
    return out_t.T


# speedup vs baseline: 3.2710x; 1.0909x over previous
import functools

import jax
import jax.numpy as jnp
from jax import lax
from jax.experimental import pallas as pl
from jax.experimental.pallas import tpu as pltpu
from jax.experimental.pallas import tpu_sc as plsc

_LANES = 16
_IDX_CHUNK = 128
_NVC = 8192
_NVC_LOG2 = 13


def _sc_pool(x_t, emb_table, B, CTX, E):
    NC, NS = 2, 16
    NW = NC * NS
    b_per_w = B // NW
    inv = jnp.float32(1.0 / CTX)

    mesh = plsc.VectorSubcoreMesh(core_axis_name="c", subcore_axis_name="s")

    @functools.partial(
        pl.kernel,
        out_type=jax.ShapeDtypeStruct((B, E), jnp.float32),
        mesh=mesh,
        scratch_types=[
            pltpu.VMEM((CTX, b_per_w), jnp.int32),
            pltpu.VMEM((b_per_w, E), jnp.float32),
            pltpu.SemaphoreType.DMA,
        ],
        compiler_params=pltpu.CompilerParams(use_tc_tiling_on_sc=False),
    )
    def pool(x_hbm, tbl_hbm, out_hbm, idx_v, acc_v, sem):
        wid = lax.axis_index("s") * NC + lax.axis_index("c")
        base = wid * b_per_w
        pltpu.sync_copy(x_hbm.at[:, pl.ds(base, b_per_w)], idx_v)
        for j in range(CTX):
            for cch in range(b_per_w // _LANES):
                sl = pl.ds(cch * _LANES, _LANES)
                v = idx_v[j, sl]
                hi = v & jnp.int32(-_NVC)
                lo = lax.shift_left(v & jnp.int32(_NVC // 2 - 1),
                                    jnp.int32(1))
                bit = lax.shift_right_logical(v, jnp.int32(_NVC_LOG2 - 1))
                idx_v[j, sl] = hi | lo | (bit & jnp.int32(1))
        zeros = jnp.zeros((_LANES,), jnp.float32)
        def zbody(i, carry):
            for cch in range(E // _LANES):
                acc_v[i, pl.ds(cch * _LANES, _LANES)] = zeros
            return carry
        lax.fori_loop(0, b_per_w, zbody, 0)
        copies = []
        for j in range(CTX):
            copies.append(pltpu.async_copy(
                tbl_hbm.at[idx_v.at[j]], acc_v, sem, add=True,
            ))
        for c in copies:
            c.wait()
        def sbody(i, carry):
            for cch in range(E // _LANES):
                sl = pl.ds(cch * _LANES, _LANES)
                acc_v[i, sl] = acc_v[i, sl] * inv
            return carry
        lax.fori_loop(0, b_per_w, sbody, 0)
        pltpu.sync_copy(acc_v, out_hbm.at[pl.ds(base, b_per_w)])

    return pool(x_t, emb_table)


def _tc_detile(emb_t, NVc=_NVC):
    E, V = emb_t.shape
    grid = pl.cdiv(V, NVc)

    def det(in_ref, o_ref):
        t = jnp.transpose(in_ref[...])
        o_ref[...] = jnp.concatenate(
            [t[0:NVc // 2], t[NVc // 2:NVc]], axis=1)

    return pl.pallas_call(
        det,
        grid=(grid,),
        in_specs=[pl.BlockSpec((E, NVc), lambda i: (0, i))],
        out_specs=pl.BlockSpec((NVc * E // 128, 128), lambda i: (i, 0)),
        out_shape=jax.ShapeDtypeStruct((grid * NVc * E // 128, 128),
                                       jnp.float32),
        compiler_params=pltpu.CompilerParams(
            dimension_semantics=("arbitrary",),
        ),
    )(emb_t)


def _tc_project_t(pooled, w_t, b_row, NV=2048):
    B, E = pooled.shape
    V = w_t.shape[1]
    grid = pl.cdiv(V, NV)

    def mm(p_ref, w_ref, b_ref, o_ref):
        dot = lax.dot_general(
            w_ref[...], p_ref[...],
            (((0,), (1,)), ((), ())),
            preferred_element_type=jnp.float32,
        )
        o_ref[...] = dot + jnp.transpose(b_ref[...])

    return pl.pallas_call(
        mm,
        grid=(grid,),
        in_specs=[
            pl.BlockSpec((B, E), lambda i: (0, 0)),
            pl.BlockSpec((E, NV), lambda i: (0, i)),
            pl.BlockSpec((1, NV), lambda i: (0, i)),
        ],
        out_specs=pl.BlockSpec((NV, B), lambda i: (i, 0)),
        out_shape=jax.ShapeDtypeStruct((V, B), jnp.float32),
        compiler_params=pltpu.CompilerParams(
            dimension_semantics=("arbitrary",),
        ),
    )(pooled, w_t, b_row)


def kernel(x, emb_table, W, b):
    B, CTX = x.shape
    V, E = emb_table.shape
    emb_lin = _tc_detile(emb_table.T).reshape(-1, E)
    pooled = _sc_pool(x.astype(jnp.int32).T, emb_lin, B, CTX, E)
    out_t = _tc_project_t(pooled, W.T, b.reshape(1, V))
    return out_t.T


# speedup vs baseline: 3.2784x; 1.0022x over previous
import functools

import jax
import jax.numpy as jnp
from jax import lax
from jax.experimental import pallas as pl
from jax.experimental.pallas import tpu as pltpu
from jax.experimental.pallas import tpu_sc as plsc

_LANES = 16
_IDX_CHUNK = 128
_NVC = 16384
_NVC_LOG2 = 14


def _sc_pool(x_t, emb_table, B, CTX, E):
    NC, NS = 2, 16
    NW = NC * NS
    b_per_w = B // NW
    inv = jnp.float32(1.0 / CTX)

    mesh = plsc.VectorSubcoreMesh(core_axis_name="c", subcore_axis_name="s")

    @functools.partial(
        pl.kernel,
        out_type=jax.ShapeDtypeStruct((B, E), jnp.float32),
        mesh=mesh,
        scratch_types=[
            pltpu.VMEM((CTX, b_per_w), jnp.int32),
            pltpu.VMEM((b_per_w, E), jnp.float32),
            pltpu.SemaphoreType.DMA,
        ],
        compiler_params=pltpu.CompilerParams(use_tc_tiling_on_sc=False),
    )
    def pool(x_hbm, tbl_hbm, out_hbm, idx_v, acc_v, sem):
        wid = lax.axis_index("s") * NC + lax.axis_index("c")
        base = wid * b_per_w
        pltpu.sync_copy(x_hbm.at[:, pl.ds(base, b_per_w)], idx_v)
        for j in range(CTX):
            for cch in range(b_per_w // _LANES):
                sl = pl.ds(cch * _LANES, _LANES)
                v = idx_v[j, sl]
                hi = v & jnp.int32(-_NVC)
                lo = lax.shift_left(v & jnp.int32(_NVC // 2 - 1),
                                    jnp.int32(1))
                bit = lax.shift_right_logical(v, jnp.int32(_NVC_LOG2 - 1))
                idx_v[j, sl] = hi | lo | (bit & jnp.int32(1))
        zeros = jnp.zeros((_LANES,), jnp.float32)
        def zbody(i, carry):
            for cch in range(E // _LANES):
                acc_v[i, pl.ds(cch * _LANES, _LANES)] = zeros
            return carry
        lax.fori_loop(0, b_per_w, zbody, 0)
        copies = []
        for j in range(CTX):
            copies.append(pltpu.async_copy(
                tbl_hbm.at[idx_v.at[j]], acc_v, sem, add=True,
            ))
        for c in copies:
            c.wait()
        def sbody(i, carry):
            for cch in range(E // _LANES):
                sl = pl.ds(cch * _LANES, _LANES)
                acc_v[i, sl] = acc_v[i, sl] * inv
            return carry
        lax.fori_loop(0, b_per_w, sbody, 0)
        pltpu.sync_copy(acc_v, out_hbm.at[pl.ds(base, b_per_w)])

    return pool(x_t, emb_table)


def _tc_detile(emb_t, NVc=_NVC):
    E, V = emb_t.shape
    grid = pl.cdiv(V, NVc)

    def det(in_ref, o_ref):
        t = jnp.transpose(in_ref[...])
        o_ref[...] = jnp.concatenate(
            [t[0:NVc // 2], t[NVc // 2:NVc]], axis=1)

    return pl.pallas_call(
        det,
        grid=(grid,),
        in_specs=[pl.BlockSpec((E, NVc), lambda i: (0, i))],
        out_specs=pl.BlockSpec((NVc * E // 128, 128), lambda i: (i, 0)),
        out_shape=jax.ShapeDtypeStruct((grid * NVc * E // 128, 128),
                                       jnp.float32),
        compiler_params=pltpu.CompilerParams(
            dimension_semantics=("arbitrary",),
        ),
    )(emb_t)


def _tc_project_t(pooled, w_t, b_row, NV=2048):
    B, E = pooled.shape
    V = w_t.shape[1]
    grid = pl.cdiv(V, NV)

    def mm(p_ref, w_ref, b_ref, o_ref):
        dot = lax.dot_general(
            w_ref[...], p_ref[...],
            (((0,), (1,)), ((), ())),
            preferred_element_type=jnp.float32,
        )
        o_ref[...] = dot + jnp.transpose(b_ref[...])

    return pl.pallas_call(
        mm,
        grid=(grid,),
        in_specs=[
            pl.BlockSpec((B, E), lambda i: (0, 0)),
            pl.BlockSpec((E, NV), lambda i: (0, i)),
            pl.BlockSpec((1, NV), lambda i: (0, i)),
        ],
        out_specs=pl.BlockSpec((NV, B), lambda i: (i, 0)),
        out_shape=jax.ShapeDtypeStruct((V, B), jnp.float32),
        compiler_params=pltpu.CompilerParams(
            dimension_semantics=("arbitrary",),
        ),
    )(pooled, w_t, b_row)


def kernel(x, emb_table, W, b):
    B, CTX = x.shape
    V, E = emb_table.shape
    emb_lin = _tc_detile(emb_table.T).reshape(-1, E)
    pooled = _sc_pool(x.astype(jnp.int32).T, emb_lin, B, CTX, E)
    out_t = _tc_project_t(pooled, W.T, b.reshape(1, V))
    return out_t.T


# speedup vs baseline: 3.3239x; 1.0139x over previous
import functools

import jax
import jax.numpy as jnp
from jax import lax
from jax.experimental import pallas as pl
from jax.experimental.pallas import tpu as pltpu
from jax.experimental.pallas import tpu_sc as plsc

_LANES = 16
_IDX_CHUNK = 128
_NVC = 16384
_NVC_LOG2 = 14


def _sc_pool(x_t, emb_table, B, CTX, E):
    NC, NS = 2, 16
    NW = NC * NS
    b_per_w = B // NW
    inv = jnp.float32(1.0 / CTX)

    mesh = plsc.VectorSubcoreMesh(core_axis_name="c", subcore_axis_name="s")

    @functools.partial(
        pl.kernel,
        out_type=jax.ShapeDtypeStruct((B, E), jnp.float32),
        mesh=mesh,
        scratch_types=[
            pltpu.VMEM((CTX, b_per_w), jnp.int32),
            pltpu.VMEM((b_per_w, E), jnp.float32),
            pltpu.SemaphoreType.DMA,
        ],
        compiler_params=pltpu.CompilerParams(use_tc_tiling_on_sc=False),
    )
    def pool(x_hbm, tbl_hbm, out_hbm, idx_v, acc_v, sem):
        wid = lax.axis_index("s") * NC + lax.axis_index("c")
        base = wid * b_per_w
        pltpu.sync_copy(x_hbm.at[:, pl.ds(base, b_per_w)], idx_v)
        for j in range(CTX):
            for cch in range(b_per_w // _LANES):
                sl = pl.ds(cch * _LANES, _LANES)
                v = idx_v[j, sl]
                hi = v & jnp.int32(-_NVC)
                lo = lax.shift_left(v & jnp.int32(_NVC // 2 - 1),
                                    jnp.int32(1))
                bit = lax.shift_right_logical(v, jnp.int32(_NVC_LOG2 - 1))
                idx_v[j, sl] = hi | lo | (bit & jnp.int32(1))
        zeros = jnp.zeros((_LANES,), jnp.float32)
        def zbody(i, carry):
            for cch in range(E // _LANES):
                acc_v[i, pl.ds(cch * _LANES, _LANES)] = zeros
            return carry
        lax.fori_loop(0, b_per_w, zbody, 0)
        copies = []
        for j in range(CTX):
            copies.append(pltpu.async_copy(
                tbl_hbm.at[idx_v.at[j]], acc_v, sem, add=True,
            ))
        for c in copies:
            c.wait()
        def sbody(i, carry):
            for cch in range(E // _LANES):
                sl = pl.ds(cch * _LANES, _LANES)
                acc_v[i, sl] = acc_v[i, sl] * inv
            return carry
        lax.fori_loop(0, b_per_w, sbody, 0)
        pltpu.sync_copy(acc_v, out_hbm.at[pl.ds(base, b_per_w)])

    return pool(x_t, emb_table)


def _tc_detile(emb_t, NVc=_NVC):
    E, V = emb_t.shape
    grid = pl.cdiv(V, NVc)

    def det(in_ref, o_ref):
        t = jnp.transpose(in_ref[...])
        o_ref[...] = jnp.concatenate(
            [t[0:NVc // 2], t[NVc // 2:NVc]], axis=1)

    return pl.pallas_call(
        det,
        grid=(grid,),
        in_specs=[pl.BlockSpec((E, NVc), lambda i: (0, i))],
        out_specs=pl.BlockSpec((NVc * E // 128, 128), lambda i: (i, 0)),
        out_shape=jax.ShapeDtypeStruct((grid * NVc * E // 128, 128),
                                       jnp.float32),
        compiler_params=pltpu.CompilerParams(
            dimension_semantics=("arbitrary",),
        ),
    )(emb_t)


def _tc_project_t(pooled, w_t, b_row, NV=4096):
    B, E = pooled.shape
    V = w_t.shape[1]
    grid = pl.cdiv(V, NV)

    def mm(p_ref, w_ref, b_ref, o_ref):
        dot = lax.dot_general(
            w_ref[...], p_ref[...],
            (((0,), (1,)), ((), ())),
            preferred_element_type=jnp.float32,
        )
        o_ref[...] = dot + jnp.transpose(b_ref[...])

    return pl.pallas_call(
        mm,
        grid=(grid,),
        in_specs=[
            pl.BlockSpec((B, E), lambda i: (0, 0)),
            pl.BlockSpec((E, NV), lambda i: (0, i)),
            pl.BlockSpec((1, NV), lambda i: (0, i)),
        ],
        out_specs=pl.BlockSpec((NV, B), lambda i: (i, 0)),
        out_shape=jax.ShapeDtypeStruct((V, B), jnp.float32),
        compiler_params=pltpu.CompilerParams(
            dimension_semantics=("arbitrary",),
        ),
    )(pooled, w_t, b_row)


def kernel(x, emb_table, W, b):
    B, CTX = x.shape
    V, E = emb_table.shape
    emb_lin = _tc_detile(emb_table.T).reshape(-1, E)
    pooled = _sc_pool(x.astype(jnp.int32).T, emb_lin, B, CTX, E)
    out_t = _tc_project_t(pooled, W.T, b.reshape(1, V))
    return out_t.T
